# Initial kernel scaffold; baseline (speedup 1.0000x reference)
import jax
import jax.numpy as jnp
from jax.experimental import pallas as pl


def kernel(segmentation, hic_matrix, features):
    raise NotImplementedError("write your pallas kernel here")


# speedup vs baseline: 5.5353x; 5.5353x over previous
import functools

import jax
import jax.numpy as jnp
from jax import lax
from jax.experimental import pallas as pl
from jax.experimental.pallas import tpu as pltpu
from jax.experimental.pallas import tpu_sc as plsc

ALPHA = 1.0
BETA = 2.0
GAMMA = 1.5
WIN = 5
B = 2
H = 512
W = 512
C = 96
K_ES = 26214
K_CP = 51
NPAIR = B * K_CP
IDX_PAD = 64
N = H * W
INF_BITS = 0x7F800000


def _roll_w(x, s):
    s = s % W
    if s == 0:
        return x
    return jnp.concatenate([x[:, W - s:], x[:, :W - s]], axis=1)


def _tc_body(seg_ref, hic_ref, part_ref, idx_ref, edge_ref):
    seg = seg_ref[0]
    hic = hic_ref[0]

    tad = (seg > 0.5).astype(jnp.float32)
    down = jnp.zeros_like(hic)
    up = jnp.zeros_like(hic)
    for s in range(1, WIN + 1):
        down = down + _roll_w(hic, s)
        up = up + _roll_w(hic, -s)
    total = up + down + 1e-8
    di = (down - up) / total
    x = di * tad
    s1 = jnp.sum(x)
    s2 = jnp.sum(x * x)
    var = (s2 - s1 * s1 / N) / (N - 1)
    part_ref[0, 0, 0] = var

    z_row = jnp.zeros((1, W + 2), dtype=jnp.float32)
    z_col = jnp.zeros((H, 1), dtype=jnp.float32)
    pad = jnp.concatenate([z_col, seg, z_col], axis=1)
    pad = jnp.concatenate([z_row, pad, z_row], axis=0)
    gx = ((pad[0:H, 2:W + 2] - pad[0:H, 0:W])
          + 2.0 * (pad[1:H + 1, 2:W + 2] - pad[1:H + 1, 0:W])
          + (pad[2:H + 2, 2:W + 2] - pad[2:H + 2, 0:W]))
    gy = ((pad[2:H + 2, 0:W] + 2.0 * pad[2:H + 2, 1:W + 1] + pad[2:H + 2, 2:W + 2])
          - (pad[0:H, 0:W] + 2.0 * pad[0:H, 1:W + 1] + pad[0:H, 2:W + 2]))
    mag = jnp.sqrt(gx * gx + gy * gy + 1e-8)
    bits = lax.bitcast_convert_type(mag, jnp.int32)

    def _bisect(k, lo, hi):
        def body(_, carry):
            lo, hi = carry
            mid = lo + (hi - lo) // 2
            cnt = jnp.sum((bits >= mid).astype(jnp.int32))
            big = cnt >= k
            return (jnp.where(big, mid, lo), jnp.where(big, hi, mid))
        lo, hi = lax.fori_loop(0, 31, body, (lo, hi))
        return lo

    t_es = _bisect(K_ES, jnp.int32(0), jnp.int32(INF_BITS))
    mask = (bits >= t_es).astype(jnp.float32)
    part_ref[0, 0, 1] = jnp.sum(mag * mask) / jnp.sum(mask)

    eh = jnp.abs(seg[1:, :] - seg[:-1, :])
    ev = jnp.abs(seg[:, 1:] - seg[:, :-1])
    eh = jnp.concatenate([eh, jnp.zeros((1, W), jnp.float32)], axis=0)
    ev = jnp.concatenate([ev, jnp.zeros((H, 1), jnp.float32)], axis=1)
    edge_ref[...] = jnp.maximum(eh, ev)

    riota = lax.broadcasted_iota(jnp.int32, (1, H), 1)
    ciota = lax.broadcasted_iota(jnp.int32, (1, W), 1)
    rowmax = jnp.max(edge_ref[...], axis=1).reshape(1, H)

    def sel_body(t, rowmax):
        gmax = jnp.max(rowmax)
        r = jnp.min(jnp.where(rowmax == gmax, riota, H))
        row = edge_ref[pl.ds(r, 1), :]
        c = jnp.min(jnp.where(row == gmax, ciota, W))
        idx_ref[0, 0, t] = r * W + c
        newrow = jnp.where(ciota == c, -1.0, row)
        edge_ref[pl.ds(r, 1), :] = newrow
        return jnp.where(riota == r, jnp.max(newrow), rowmax)

    lax.fori_loop(0, K_CP, sel_body, rowmax)
    for t in range(K_CP, IDX_PAD):
        idx_ref[0, 0, t] = 0
    part_ref[0, 0, 2] = 0.0
    part_ref[0, 0, 3] = 0.0


def _tc_stats(seg, hic):
    return pl.pallas_call(
        _tc_body,
        grid=(B,),
        in_specs=[
            pl.BlockSpec((1, H, W), lambda b: (b, 0, 0)),
            pl.BlockSpec((1, H, W), lambda b: (b, 0, 0)),
        ],
        out_specs=[
            pl.BlockSpec((1, 1, 4), lambda b: (b, 0, 0), memory_space=pltpu.SMEM),
            pl.BlockSpec((1, 1, IDX_PAD), lambda b: (b, 0, 0), memory_space=pltpu.SMEM),
        ],
        out_shape=[
            jax.ShapeDtypeStruct((B, 1, 4), jnp.float32),
            jax.ShapeDtypeStruct((B, 1, IDX_PAD), jnp.int32),
        ],
        scratch_shapes=[pltpu.VMEM((H, W), jnp.float32)],
    )(seg, hic)


CH_PASS = 32
NPASS = C // CH_PASS
ROWS_PER_PASS = CH_PASS * 10 * 2
GCHUNK = 128
NCHUNK = ROWS_PER_PASS // GCHUNK


def _nsqrt(v):
    b0 = lax.bitcast_convert_type(v, jnp.int32)
    y = lax.bitcast_convert_type(
        jnp.int32(0x1FBD1DF5) + lax.shift_right_logical(b0, 1), jnp.float32)
    for _ in range(3):
        y = 0.5 * (y + v / y)
    return y


def _sc_body(feat_hbm, idx_hbm, out_hbm, idxbuf, idxg, strip, scorebuf, sem):
    info = plsc.get_sparse_core_info()
    nc = info.num_cores
    wid = lax.axis_index("s") * nc + lax.axis_index("c")
    lanes = lax.iota(jnp.int32, 16)

    def pair_body(t, carry):
        p = wid + 32 * t

        @pl.when(p < NPAIR)
        def _():
            b = p // K_CP
            k = p - b * K_CP
            kbase = (k // 16) * 16
            pltpu.sync_copy(idx_hbm.at[b, pl.ds(kbase, 16)], idxbuf)
            flat = plsc.load_gather(idxbuf, [lanes * 0 + (k - kbase)])
            i = flat // W
            j = flat - i * W
            validf = jnp.where(
                (i >= WIN) & (i < H - WIN) & (j >= WIN) & (j < W - WIN),
                jnp.float32(1.0), jnp.float32(0.0))
            icl = jnp.clip(i, WIN, H - WIN - 1)
            jcl = jnp.clip(j, WIN, W - WIN - 1)
            qa = (jcl - WIN) // 128
            off = (jcl - WIN) - qa * 128

            total = jnp.zeros((16,), jnp.float32)
            for ps in range(NPASS):
                base = (b * C + ps * CH_PASS) * (H * 4) + (icl - WIN) * 4

                def build(it, _):
                    n = it * 16 + lanes
                    cc = n // 20
                    rem = n - cc * 20
                    r = rem // 2
                    blk = rem - r * 2
                    rows = base + cc * (H * 4) + r * 4 + jnp.minimum(qa + blk, 3)
                    idxg[pl.ds(it * 16, 16)] = rows
                    return 0
                lax.fori_loop(0, ROWS_PER_PASS // 16, build, 0)

                copies = [
                    pltpu.async_copy(
                        feat_hbm.at[idxg.at[pl.ds(ch * GCHUNK, GCHUNK)]],
                        strip.at[pl.ds(ch * GCHUNK, GCHUNK)],
                        sem,
                    )
                    for ch in range(NCHUNK)
                ]
                for cp_ in copies:
                    cp_.wait()

                zeros = jnp.zeros((16,), jnp.float32)

                def stat_body(ppos, carry):
                    isaft = ppos >= 25
                    tt = jnp.where(isaft, ppos - 25, ppos)
                    pr = tt // 5
                    pc = tt - pr * 5
                    shift = jnp.where(isaft, 5, 0)
                    r = pr + shift
                    colpos = off + pc + shift
                    cb = colpos // 128
                    cr = colpos - cb * 128
                    mb = jnp.where(isaft, jnp.float32(0.0), jnp.float32(1.0))
                    ma = jnp.float32(1.0) - mb

                    out = []
                    for g in range(CH_PASS // 16):
                        bsum, bss, asum, ass = carry[g]
                        nrow = (g * 16 + lanes) * 20 + r * 2 + cb
                        val = plsc.load_gather(strip, [nrow, cr])
                        v2 = val * val
                        out.append((bsum + val * mb, bss + v2 * mb,
                                    asum + val * ma, ass + v2 * ma))
                    return tuple(out)

                accs = lax.fori_loop(
                    0, 50, stat_body,
                    tuple((zeros, zeros, zeros, zeros) for _ in range(CH_PASS // 16)))

                for g in range(CH_PASS // 16):
                    bsum, bss, asum, ass = accs[g]
                    bm = bsum * (1.0 / 25.0)
                    am = asum * (1.0 / 25.0)
                    bvar = jnp.maximum((bss - bsum * bsum * (1.0 / 25.0)) * (1.0 / 24.0), 0.0)
                    avar = jnp.maximum((ass - asum * asum * (1.0 / 25.0)) * (1.0 / 24.0), 0.0)
                    bs = _nsqrt(bvar) + 1e-8
                    a_s = _nsqrt(avar) + 1e-8
                    total = total + jnp.abs(am - bm) / (bs + a_s)

            scorebuf[...] = total
            acc = jnp.zeros((16,), jnp.float32)
            for l in range(16):
                acc = acc + plsc.load_gather(scorebuf, [lanes * 0 + l])
            scorebuf[...] = acc * (1.0 / C) * validf
            pltpu.sync_copy(scorebuf, out_hbm.at[p])
        return 0

    lax.fori_loop(0, 4, pair_body, 0)


def _sc_changepoint(featv, idx):
    mesh = plsc.VectorSubcoreMesh(core_axis_name="c", subcore_axis_name="s")
    kfn = functools.partial(
        pl.kernel,
        _sc_body,
        out_type=jax.ShapeDtypeStruct((128, 16), jnp.float32),
        mesh=mesh,
        compiler_params=pltpu.CompilerParams(needs_layout_passes=False),
        scratch_types=[
            pltpu.VMEM((16,), jnp.int32),
            pltpu.VMEM((ROWS_PER_PASS,), jnp.int32),
            pltpu.VMEM((ROWS_PER_PASS, 128), jnp.float32),
            pltpu.VMEM((16,), jnp.float32),
            pltpu.SemaphoreType.DMA,
        ],
    )()
    return kfn(featv, idx)


def kernel(segmentation, hic_matrix, features):
    seg = segmentation.reshape(B, H, W)
    hic = hic_matrix.reshape(B, H, W)
    part, idx = _tc_stats(seg, hic)
    part = part.reshape(B, 4)
    idx = idx.reshape(B, IDX_PAD)

    ic = part[:, 0].mean()
    es = part[:, 1].mean()

    featv = features.reshape(-1, 128)
    scores_all = _sc_changepoint(featv, idx)
    sc = scores_all[:NPAIR, 0].reshape(B, K_CP)

    fi = idx[:, :K_CP]
    ii = fi // W
    jj = fi % W
    valid = ((ii >= WIN) & (ii < H - WIN) & (jj >= WIN) & (jj < W - WIN))
    cnt = valid.sum(axis=1).astype(jnp.float32)
    cp_b = jnp.where(cnt > 0, sc.sum(axis=1) / jnp.maximum(cnt, 1.0), 0.0)
    cp = cp_b.mean()

    return ALPHA * ic - BETA * es + GAMMA * cp


# speedup vs baseline: 21.7842x; 3.9355x over previous
import functools

import jax
import jax.numpy as jnp
from jax import lax
from jax.experimental import pallas as pl
from jax.experimental.pallas import tpu as pltpu
from jax.experimental.pallas import tpu_sc as plsc

ALPHA = 1.0
BETA = 2.0
GAMMA = 1.5
WIN = 5
B = 2
H = 512
W = 512
C = 96
K_ES = 26214
K_CP = 51
NPAIR = B * K_CP
IDX_PAD = 64
N = H * W
INF_BITS = 0x7F800000


def _roll_w(x, s):
    s = s % W
    if s == 0:
        return x
    return jnp.concatenate([x[:, W - s:], x[:, :W - s]], axis=1)


def _cumsum_lanes(x):
    rows, width = x.shape
    sh = 1
    while sh < width:
        z = jnp.zeros((rows, sh), jnp.float32)
        x = x + jnp.concatenate([z, x[:, :width - sh]], axis=1)
        sh *= 2
    return x


def _tc_sel_body(seg_ref, idx_ref):
    seg = seg_ref[0]

    eh = jnp.abs(seg[1:, :] - seg[:-1, :])
    ev = jnp.abs(seg[:, 1:] - seg[:, :-1])
    eh = jnp.concatenate([eh, jnp.zeros((1, W), jnp.float32)], axis=0)
    ev = jnp.concatenate([ev, jnp.zeros((H, 1), jnp.float32)], axis=1)
    edge = jnp.maximum(eh, ev)
    bits = lax.bitcast_convert_type(edge, jnp.int32)

    def body(_, carry):
        lo, hi = carry
        span = hi - lo
        m1 = lo + span // 4
        m2 = lo + span // 2
        m3 = lo + span // 2 + span // 4
        c1 = jnp.sum((bits >= m1).astype(jnp.int32))
        c2 = jnp.sum((bits >= m2).astype(jnp.int32))
        c3 = jnp.sum((bits >= m3).astype(jnp.int32))
        ge1 = c1 >= K_CP
        ge2 = c2 >= K_CP
        ge3 = c3 >= K_CP
        lo2 = jnp.where(ge3, m3, jnp.where(ge2, m2, jnp.where(ge1, m1, lo)))
        hi2 = jnp.where(ge3, hi, jnp.where(ge2, m3, jnp.where(ge1, m2, m1)))
        return (lo2, hi2)
    t51, _ = lax.fori_loop(0, 16, body, (jnp.int32(0), jnp.int32(INF_BITS)))

    gt = (bits > t51).astype(jnp.float32)
    eq = (bits == t51).astype(jnp.float32)
    c1 = jnp.sum(gt)
    need = jnp.float32(K_CP) - c1

    riota_col = lax.broadcasted_iota(jnp.int32, (H, 1), 0)
    tril = (lax.broadcasted_iota(jnp.int32, (H, H), 1)
            < lax.broadcasted_iota(jnp.int32, (H, H), 0)).astype(jnp.float32)

    eq_cum = _cumsum_lanes(eq)
    eq_rowoff = jnp.dot(tril, eq_cum[:, W - 1:W],
                        preferred_element_type=jnp.float32)
    eq_rank = eq_cum + eq_rowoff
    sel = jnp.maximum(gt, eq * (eq_rank <= need).astype(jnp.float32))

    sel_cum = _cumsum_lanes(sel)
    row_cnt = sel_cum[:, W - 1:W]
    rowoff = jnp.dot(tril, row_cnt, preferred_element_type=jnp.float32)

    srow = lax.broadcasted_iota(jnp.int32, (IDX_PAD, H), 0).astype(jnp.float32) + 1.0
    ro_b = rowoff.reshape(1, H)
    rc_b = row_cnt.reshape(1, H)
    ind = ((srow > ro_b) & (srow <= ro_b + rc_b)).astype(jnp.float32)

    r_s = jnp.dot(ind, riota_col.astype(jnp.float32),
                  preferred_element_type=jnp.float32)
    ro_s = jnp.dot(ind, rowoff, preferred_element_type=jnp.float32)
    g_sel = jnp.dot(ind, sel, preferred_element_type=jnp.float32)
    g_cum = _cumsum_lanes(g_sel)

    sprime = (lax.broadcasted_iota(jnp.int32, (IDX_PAD, 1), 0).astype(jnp.float32)
              + 1.0 - ro_s)
    ciota64 = lax.broadcasted_iota(jnp.int32, (IDX_PAD, W), 1)
    hit = (g_sel > 0.5) & (g_cum == sprime)
    c_s = jnp.min(jnp.where(hit, ciota64, W), axis=1, keepdims=True)

    flat = r_s.astype(jnp.int32) * W + c_s
    idx_ref[0, 0, :] = flat.reshape(1, IDX_PAD)[0, :]


def _tc_stats_body(seg_ref, hic_ref, part_ref):
    seg = seg_ref[0]
    hic = hic_ref[0]

    tad = (seg > 0.5).astype(jnp.float32)
    down = jnp.zeros_like(hic)
    up = jnp.zeros_like(hic)
    for s in range(1, WIN + 1):
        down = down + _roll_w(hic, s)
        up = up + _roll_w(hic, -s)
    total = up + down + 1e-8
    di = (down - up) / total
    x = di * tad
    s1 = jnp.sum(x)
    s2 = jnp.sum(x * x)
    var = (s2 - s1 * s1 / N) / (N - 1)
    part_ref[0, 0, 0] = var

    z_row = jnp.zeros((1, W + 2), dtype=jnp.float32)
    z_col = jnp.zeros((H, 1), dtype=jnp.float32)
    pad = jnp.concatenate([z_col, seg, z_col], axis=1)
    pad = jnp.concatenate([z_row, pad, z_row], axis=0)
    gx = ((pad[0:H, 2:W + 2] - pad[0:H, 0:W])
          + 2.0 * (pad[1:H + 1, 2:W + 2] - pad[1:H + 1, 0:W])
          + (pad[2:H + 2, 2:W + 2] - pad[2:H + 2, 0:W]))
    gy = ((pad[2:H + 2, 0:W] + 2.0 * pad[2:H + 2, 1:W + 1] + pad[2:H + 2, 2:W + 2])
          - (pad[0:H, 0:W] + 2.0 * pad[0:H, 1:W + 1] + pad[0:H, 2:W + 2]))
    mag = jnp.sqrt(gx * gx + gy * gy + 1e-8)
    bits = lax.bitcast_convert_type(mag, jnp.int32)

    def _bisect(k, lo, hi, iters):
        def body(_, carry):
            lo, hi = carry
            mid = lo + (hi - lo) // 2
            cnt = jnp.sum((bits >= mid).astype(jnp.int32))
            big = cnt >= k
            return (jnp.where(big, mid, lo), jnp.where(big, hi, mid))
        lo, hi = lax.fori_loop(0, iters, body, (lo, hi))
        return lo

    t_es = _bisect(K_ES, jnp.int32(0), jnp.int32(INF_BITS), 31)
    mask = (bits >= t_es).astype(jnp.float32)
    part_ref[0, 0, 1] = jnp.sum(mag * mask) / jnp.sum(mask)
    part_ref[0, 0, 2] = 0.0
    part_ref[0, 0, 3] = 0.0


def _tc_sel(seg):
    return pl.pallas_call(
        _tc_sel_body,
        grid=(B,),
        in_specs=[pl.BlockSpec((1, H, W), lambda b: (b, 0, 0))],
        out_specs=pl.BlockSpec((1, 1, IDX_PAD), lambda b: (b, 0, 0)),
        out_shape=jax.ShapeDtypeStruct((B, 1, IDX_PAD), jnp.int32),
    )(seg)


def _tc_stats(seg, hic):
    return pl.pallas_call(
        _tc_stats_body,
        grid=(B,),
        in_specs=[
            pl.BlockSpec((1, H, W), lambda b: (b, 0, 0)),
            pl.BlockSpec((1, H, W), lambda b: (b, 0, 0)),
        ],
        out_specs=pl.BlockSpec((1, 1, 4), lambda b: (b, 0, 0),
                               memory_space=pltpu.SMEM),
        out_shape=jax.ShapeDtypeStruct((B, 1, 4), jnp.float32),
    )(seg, hic)


CH_PASS = 16
NPASS = C // CH_PASS
ROWS_PER_PASS = CH_PASS * 10
GCHUNK = 80
NCHUNK = ROWS_PER_PASS // GCHUNK


def _nsqrt(v):
    b0 = lax.bitcast_convert_type(v, jnp.int32)
    y = lax.bitcast_convert_type(
        jnp.int32(0x1FBD1DF5) + lax.shift_right_logical(b0, 1), jnp.float32)
    for _ in range(3):
        y = 0.5 * (y + v / y)
    return y


def _sc_body(feat_hbm, idx_hbm, out_hbm, idxbuf, idxg0, idxg1, strip0, strip1,
             scorebuf, sem0, sem1):
    info = plsc.get_sparse_core_info()
    nc = info.num_cores
    wid = lax.axis_index("s") * nc + lax.axis_index("c")
    lanes = lax.iota(jnp.int32, 16)
    sems = (sem0, sem1)
    idxgs = (idxg0, idxg1)
    strips = (strip0, strip1)

    def pair_body(t, carry):
        p = wid + 32 * t

        @pl.when(p < NPAIR)
        def _():
            b = p // K_CP
            k = p - b * K_CP
            kbase = (k // 16) * 16
            pltpu.sync_copy(idx_hbm.at[b, pl.ds(kbase, 16)], idxbuf)
            flat = jnp.max(jnp.where(lanes == (k - kbase), idxbuf[...], 0))
            i = flat // W
            j = flat - i * W
            validf = jnp.where(
                (i >= WIN) & (i < H - WIN) & (j >= WIN) & (j < W - WIN),
                jnp.float32(1.0), jnp.float32(0.0))
            icl = jnp.clip(i, WIN, H - WIN - 1)
            jcl = jnp.clip(j, WIN, W - WIN - 1)
            qa = jnp.minimum((jcl - WIN) // 128, 2)
            q0 = qa * 128
            j0 = (jcl - WIN) - q0
            crossing = j0 > 118

            def start_pass(ps, buf):
                base = (b * C + ps * CH_PASS) * H + (icl - WIN)

                idxg = idxgs[buf]
                strip = strips[buf]

                def build(it, _):
                    n = it * 16 + lanes
                    cc = n // 10
                    r = n - cc * 10
                    idxg[pl.ds(it * 16, 16)] = base + cc * H + r
                    return 0
                lax.fori_loop(0, ROWS_PER_PASS // 16, build, 0)

                @pl.when(crossing)
                def _():
                    for ch in range(NCHUNK):
                        pltpu.async_copy(
                            feat_hbm.at[idxg.at[pl.ds(ch * GCHUNK, GCHUNK)],
                                        pl.ds(q0, 256)],
                            strip.at[pl.ds(ch * GCHUNK, GCHUNK)],
                            sems[buf],
                        )

                @pl.when(jnp.logical_not(crossing))
                def _():
                    for ch in range(NCHUNK):
                        pltpu.async_copy(
                            feat_hbm.at[idxg.at[pl.ds(ch * GCHUNK, GCHUNK)],
                                        pl.ds(q0, 128)],
                            strip.at[pl.ds(ch * GCHUNK, GCHUNK), pl.ds(0, 128)],
                            sems[buf],
                        )

            def drain_pass(buf):

                @pl.when(crossing)
                def _():
                    for ch in range(NCHUNK):
                        pltpu.make_async_copy(
                            feat_hbm.at[idxgs[buf].at[pl.ds(ch * GCHUNK, GCHUNK)],
                                        pl.ds(q0, 256)],
                            strips[buf].at[pl.ds(ch * GCHUNK, GCHUNK)],
                            sems[buf],
                        ).wait()

                @pl.when(jnp.logical_not(crossing))
                def _():
                    for ch in range(NCHUNK):
                        pltpu.make_async_copy(
                            feat_hbm.at[idxgs[buf].at[pl.ds(ch * GCHUNK, GCHUNK)],
                                        pl.ds(q0, 128)],
                            strips[buf].at[pl.ds(ch * GCHUNK, GCHUNK), pl.ds(0, 128)],
                            sems[buf],
                        ).wait()

            start_pass(0, 0)
            total = jnp.zeros((16,), jnp.float32)
            zeros = jnp.zeros((16,), jnp.float32)
            for psl in range(NPASS):
                if psl + 1 < NPASS:
                    start_pass(psl + 1, (psl + 1) % 2)
                drain_pass(psl % 2)
                buf = psl % 2

                def stat_body(ppos, carry):
                    bsum, bss, asum, ass = carry
                    isaft = ppos >= 25
                    tt = jnp.where(isaft, ppos - 25, ppos)
                    pr = tt // 5
                    pc = tt - pr * 5
                    shift = jnp.where(isaft, 5, 0)
                    nrow = lanes * 10 + pr + shift
                    ncol = lanes * 0 + (j0 + pc + shift)
                    mb = jnp.where(isaft, jnp.float32(0.0), jnp.float32(1.0))
                    ma = jnp.float32(1.0) - mb
                    val = plsc.load_gather(strips[buf], [nrow, ncol])
                    v2 = val * val
                    return (bsum + val * mb, bss + v2 * mb,
                            asum + val * ma, ass + v2 * ma)

                bsum, bss, asum, ass = lax.fori_loop(
                    0, 50, stat_body, (zeros, zeros, zeros, zeros), unroll=5)

                bm = bsum * (1.0 / 25.0)
                am = asum * (1.0 / 25.0)
                bvar = jnp.maximum((bss - bsum * bsum * (1.0 / 25.0)) * (1.0 / 24.0), 0.0)
                avar = jnp.maximum((ass - asum * asum * (1.0 / 25.0)) * (1.0 / 24.0), 0.0)
                bs = _nsqrt(bvar) + 1e-8
                a_s = _nsqrt(avar) + 1e-8
                total = total + jnp.abs(am - bm) / (bs + a_s)

            scorebuf[...] = total
            acc = jnp.zeros((16,), jnp.float32)
            for l in range(16):
                acc = acc + plsc.load_gather(scorebuf, [lanes * 0 + l])
            scorebuf[...] = acc * (1.0 / C) * validf
            pltpu.sync_copy(scorebuf, out_hbm.at[p])
        return 0

    lax.fori_loop(0, 4, pair_body, 0)


def _sc_changepoint(featv, idx):
    mesh = plsc.VectorSubcoreMesh(core_axis_name="c", subcore_axis_name="s")
    kfn = functools.partial(
        pl.kernel,
        _sc_body,
        out_type=jax.ShapeDtypeStruct((128, 16), jnp.float32),
        mesh=mesh,
        compiler_params=pltpu.CompilerParams(needs_layout_passes=False),
        scratch_types=[
            pltpu.VMEM((16,), jnp.int32),
            pltpu.VMEM((ROWS_PER_PASS,), jnp.int32),
            pltpu.VMEM((ROWS_PER_PASS,), jnp.int32),
            pltpu.VMEM((ROWS_PER_PASS, 256), jnp.float32),
            pltpu.VMEM((ROWS_PER_PASS, 256), jnp.float32),
            pltpu.VMEM((16,), jnp.float32),
            pltpu.SemaphoreType.DMA,
            pltpu.SemaphoreType.DMA,
        ],
    )()
    return kfn(featv, idx)


def _combine_body(part_ref, idx_ref, sc_ref, out_ref):
    ic = (part_ref[0, 0, 0] + part_ref[1, 0, 0]) * 0.5
    es = (part_ref[0, 0, 1] + part_ref[1, 0, 1]) * 0.5

    idxv = idx_ref[...]
    kio = lax.broadcasted_iota(jnp.int32, (B, 1, IDX_PAD), 2)
    ii = idxv // W
    jj = idxv - ii * W
    validm = ((kio < K_CP) & (ii >= WIN) & (ii < H - WIN)
              & (jj >= WIN) & (jj < W - WIN)).astype(jnp.float32)
    bio = lax.broadcasted_iota(jnp.int32, (B, 1, IDX_PAD), 0)
    cnt0 = jnp.sum(validm * (bio == 0))
    cnt1 = jnp.sum(validm * (bio == 1))

    col = sc_ref[:, 0:1]
    rio = lax.broadcasted_iota(jnp.int32, (128, 1), 0)
    s0 = jnp.sum(jnp.where(rio < K_CP, col, 0.0))
    s1 = jnp.sum(jnp.where((rio >= K_CP) & (rio < NPAIR), col, 0.0))

    cp0 = jnp.where(cnt0 > 0, s0 / jnp.maximum(cnt0, 1.0), 0.0)
    cp1 = jnp.where(cnt1 > 0, s1 / jnp.maximum(cnt1, 1.0), 0.0)
    cp = (cp0 + cp1) * 0.5
    out_ref[0, 0] = ALPHA * ic - BETA * es + GAMMA * cp


def _combine(part, idx, scores_all):
    return pl.pallas_call(
        _combine_body,
        in_specs=[
            pl.BlockSpec(memory_space=pltpu.SMEM),
            pl.BlockSpec((B, 1, IDX_PAD), lambda: (0, 0, 0)),
            pl.BlockSpec((128, 16), lambda: (0, 0)),
        ],
        out_specs=pl.BlockSpec(memory_space=pltpu.SMEM),
        out_shape=jax.ShapeDtypeStruct((1, 1), jnp.float32),
    )(part, idx, scores_all)


def kernel(segmentation, hic_matrix, features):
    seg = segmentation.reshape(B, H, W)
    hic = hic_matrix.reshape(B, H, W)

    idx3 = _tc_sel(seg)
    featv = features.reshape(-1, W)
    scores_all = _sc_changepoint(featv, idx3.reshape(B, IDX_PAD))
    part = _tc_stats(seg, hic)

    return _combine(part, idx3, scores_all)[0, 0]


# speedup vs baseline: 22.6602x; 1.0402x over previous
import functools

import jax
import jax.numpy as jnp
from jax import lax
from jax.experimental import pallas as pl
from jax.experimental.pallas import tpu as pltpu
from jax.experimental.pallas import tpu_sc as plsc

ALPHA = 1.0
BETA = 2.0
GAMMA = 1.5
WIN = 5
B = 2
H = 512
W = 512
C = 96
K_ES = 26214
K_CP = 51
NPAIR = B * K_CP
IDX_PAD = 64
N = H * W
INF_BITS = 0x7F800000


def _roll_w(x, s):
    s = s % W
    if s == 0:
        return x
    return jnp.concatenate([x[:, W - s:], x[:, :W - s]], axis=1)


def _cumsum_lanes(x):
    rows, width = x.shape
    sh = 1
    while sh < width:
        z = jnp.zeros((rows, sh), jnp.float32)
        x = x + jnp.concatenate([z, x[:, :width - sh]], axis=1)
        sh *= 2
    return x


def _tc_sel_body(seg_ref, idx_ref):
    seg = seg_ref[0]

    eh = jnp.abs(seg[1:, :] - seg[:-1, :])
    ev = jnp.abs(seg[:, 1:] - seg[:, :-1])
    eh = jnp.concatenate([eh, jnp.zeros((1, W), jnp.float32)], axis=0)
    ev = jnp.concatenate([ev, jnp.zeros((H, 1), jnp.float32)], axis=1)
    edge = jnp.maximum(eh, ev)
    bits = lax.bitcast_convert_type(edge, jnp.int32)

    def body(_, carry):
        lo, hi = carry
        span = hi - lo
        m1 = lo + span // 4
        m2 = lo + span // 2
        m3 = lo + span // 2 + span // 4
        c1 = jnp.sum((bits >= m1).astype(jnp.int32))
        c2 = jnp.sum((bits >= m2).astype(jnp.int32))
        c3 = jnp.sum((bits >= m3).astype(jnp.int32))
        ge1 = c1 >= K_CP
        ge2 = c2 >= K_CP
        ge3 = c3 >= K_CP
        lo2 = jnp.where(ge3, m3, jnp.where(ge2, m2, jnp.where(ge1, m1, lo)))
        hi2 = jnp.where(ge3, hi, jnp.where(ge2, m3, jnp.where(ge1, m2, m1)))
        return (lo2, hi2)
    t51, _ = lax.fori_loop(0, 16, body, (jnp.int32(0), jnp.int32(INF_BITS)))

    gt = (bits > t51).astype(jnp.float32)
    eq = (bits == t51).astype(jnp.float32)
    c1 = jnp.sum(gt)
    need = jnp.float32(K_CP) - c1

    riota_col = lax.broadcasted_iota(jnp.int32, (H, 1), 0)
    tril = (lax.broadcasted_iota(jnp.int32, (H, H), 1)
            < lax.broadcasted_iota(jnp.int32, (H, H), 0)).astype(jnp.float32)

    eq_cum = _cumsum_lanes(eq)
    eq_rowoff = jnp.dot(tril, jnp.sum(eq, axis=1, keepdims=True),
                        preferred_element_type=jnp.float32)
    eq_rank = eq_cum + eq_rowoff
    sel = jnp.maximum(gt, eq * (eq_rank <= need).astype(jnp.float32))

    row_cnt = jnp.sum(sel, axis=1, keepdims=True)
    rowoff = jnp.dot(tril, row_cnt, preferred_element_type=jnp.float32)

    srow = lax.broadcasted_iota(jnp.int32, (IDX_PAD, H), 0).astype(jnp.float32) + 1.0
    ro_b = rowoff.reshape(1, H)
    rc_b = row_cnt.reshape(1, H)
    ind = ((srow > ro_b) & (srow <= ro_b + rc_b)).astype(jnp.float32)

    r_s = jnp.dot(ind, riota_col.astype(jnp.float32),
                  preferred_element_type=jnp.float32)
    ro_s = jnp.dot(ind, rowoff, preferred_element_type=jnp.float32)
    g_sel = jnp.dot(ind, sel, preferred_element_type=jnp.float32)
    g_cum = _cumsum_lanes(g_sel)

    sprime = (lax.broadcasted_iota(jnp.int32, (IDX_PAD, 1), 0).astype(jnp.float32)
              + 1.0 - ro_s)
    ciota64 = lax.broadcasted_iota(jnp.int32, (IDX_PAD, W), 1)
    hit = (g_sel > 0.5) & (g_cum == sprime)
    c_s = jnp.min(jnp.where(hit, ciota64, W), axis=1, keepdims=True)

    flat = r_s.astype(jnp.int32) * W + c_s
    idx_ref[0, 0, :] = flat.reshape(1, IDX_PAD)[0, :]


def _tc_stats_body(seg_ref, hic_ref, part_ref):
    seg = seg_ref[0]
    hic = hic_ref[0]

    tad = (seg > 0.5).astype(jnp.float32)
    down = jnp.zeros_like(hic)
    up = jnp.zeros_like(hic)
    for s in range(1, WIN + 1):
        down = down + _roll_w(hic, s)
        up = up + _roll_w(hic, -s)
    total = up + down + 1e-8
    di = (down - up) / total
    x = di * tad
    s1 = jnp.sum(x)
    s2 = jnp.sum(x * x)
    var = (s2 - s1 * s1 / N) / (N - 1)
    part_ref[0, 0, 0] = var

    z_row = jnp.zeros((1, W + 2), dtype=jnp.float32)
    z_col = jnp.zeros((H, 1), dtype=jnp.float32)
    pad = jnp.concatenate([z_col, seg, z_col], axis=1)
    pad = jnp.concatenate([z_row, pad, z_row], axis=0)
    gx = ((pad[0:H, 2:W + 2] - pad[0:H, 0:W])
          + 2.0 * (pad[1:H + 1, 2:W + 2] - pad[1:H + 1, 0:W])
          + (pad[2:H + 2, 2:W + 2] - pad[2:H + 2, 0:W]))
    gy = ((pad[2:H + 2, 0:W] + 2.0 * pad[2:H + 2, 1:W + 1] + pad[2:H + 2, 2:W + 2])
          - (pad[0:H, 0:W] + 2.0 * pad[0:H, 1:W + 1] + pad[0:H, 2:W + 2]))
    mag = jnp.sqrt(gx * gx + gy * gy + 1e-8)
    bits = lax.bitcast_convert_type(mag, jnp.int32)

    def _bisect(k, lo, hi, iters):
        def body(_, carry):
            lo, hi = carry
            mid = lo + (hi - lo) // 2
            cnt = jnp.sum((bits >= mid).astype(jnp.int32))
            big = cnt >= k
            return (jnp.where(big, mid, lo), jnp.where(big, hi, mid))
        lo, hi = lax.fori_loop(0, iters, body, (lo, hi))
        return lo

    t_es = _bisect(K_ES, jnp.int32(0), jnp.int32(INF_BITS), 31)
    mask = (bits >= t_es).astype(jnp.float32)
    part_ref[0, 0, 1] = jnp.sum(mag * mask) / jnp.sum(mask)
    part_ref[0, 0, 2] = 0.0
    part_ref[0, 0, 3] = 0.0


def _tc_sel(seg):
    return pl.pallas_call(
        _tc_sel_body,
        grid=(B,),
        in_specs=[pl.BlockSpec((1, H, W), lambda b: (b, 0, 0))],
        out_specs=pl.BlockSpec((1, 1, IDX_PAD), lambda b: (b, 0, 0)),
        out_shape=jax.ShapeDtypeStruct((B, 1, IDX_PAD), jnp.int32),
    )(seg)


def _tc_stats(seg, hic):
    return pl.pallas_call(
        _tc_stats_body,
        grid=(B,),
        in_specs=[
            pl.BlockSpec((1, H, W), lambda b: (b, 0, 0)),
            pl.BlockSpec((1, H, W), lambda b: (b, 0, 0)),
        ],
        out_specs=pl.BlockSpec((1, 1, 4), lambda b: (b, 0, 0),
                               memory_space=pltpu.SMEM),
        out_shape=jax.ShapeDtypeStruct((B, 1, 4), jnp.float32),
    )(seg, hic)


CH_PASS = 16
NPASS = C // CH_PASS
ROWS_PER_PASS = CH_PASS * 10
GCHUNK = 80
NCHUNK = ROWS_PER_PASS // GCHUNK


def _nsqrt(v):
    b0 = lax.bitcast_convert_type(v, jnp.int32)
    y = lax.bitcast_convert_type(
        jnp.int32(0x1FBD1DF5) + lax.shift_right_logical(b0, 1), jnp.float32)
    for _ in range(3):
        y = 0.5 * (y + v / y)
    return y


def _sc_body(feat_hbm, idx_hbm, out_hbm, idxbuf, idxg0, idxg1, strip0, strip1,
             scorebuf, sem0, sem1):
    info = plsc.get_sparse_core_info()
    nc = info.num_cores
    wid = lax.axis_index("s") * nc + lax.axis_index("c")
    lanes = lax.iota(jnp.int32, 16)
    sems = (sem0, sem1)
    idxgs = (idxg0, idxg1)
    strips = (strip0, strip1)

    def pair_body(t, carry):
        p = wid + 32 * t

        @pl.when(p < NPAIR)
        def _():
            b = p // K_CP
            k = p - b * K_CP
            kbase = (k // 16) * 16
            pltpu.sync_copy(idx_hbm.at[b, pl.ds(kbase, 16)], idxbuf)
            flat = jnp.max(jnp.where(lanes == (k - kbase), idxbuf[...], 0))
            i = flat // W
            j = flat - i * W
            validf = jnp.where(
                (i >= WIN) & (i < H - WIN) & (j >= WIN) & (j < W - WIN),
                jnp.float32(1.0), jnp.float32(0.0))
            icl = jnp.clip(i, WIN, H - WIN - 1)
            jcl = jnp.clip(j, WIN, W - WIN - 1)
            qa = jnp.minimum((jcl - WIN) // 128, 2)
            q0 = qa * 128
            j0 = (jcl - WIN) - q0
            crossing = j0 > 118

            def start_pass(ps, buf):
                base = (b * C + ps * CH_PASS) * H + (icl - WIN)

                idxg = idxgs[buf]
                strip = strips[buf]

                def build(it, _):
                    n = it * 16 + lanes
                    cc = n // 10
                    r = n - cc * 10
                    idxg[pl.ds(it * 16, 16)] = base + cc * H + r
                    return 0
                lax.fori_loop(0, ROWS_PER_PASS // 16, build, 0)

                @pl.when(crossing)
                def _():
                    for ch in range(NCHUNK):
                        pltpu.async_copy(
                            feat_hbm.at[idxg.at[pl.ds(ch * GCHUNK, GCHUNK)],
                                        pl.ds(q0, 256)],
                            strip.at[pl.ds(ch * GCHUNK, GCHUNK)],
                            sems[buf],
                        )

                @pl.when(jnp.logical_not(crossing))
                def _():
                    for ch in range(NCHUNK):
                        pltpu.async_copy(
                            feat_hbm.at[idxg.at[pl.ds(ch * GCHUNK, GCHUNK)],
                                        pl.ds(q0, 128)],
                            strip.at[pl.ds(ch * GCHUNK, GCHUNK), pl.ds(0, 128)],
                            sems[buf],
                        )

            def drain_pass(buf):

                @pl.when(crossing)
                def _():
                    for ch in range(NCHUNK):
                        pltpu.make_async_copy(
                            feat_hbm.at[idxgs[buf].at[pl.ds(ch * GCHUNK, GCHUNK)],
                                        pl.ds(q0, 256)],
                            strips[buf].at[pl.ds(ch * GCHUNK, GCHUNK)],
                            sems[buf],
                        ).wait()

                @pl.when(jnp.logical_not(crossing))
                def _():
                    for ch in range(NCHUNK):
                        pltpu.make_async_copy(
                            feat_hbm.at[idxgs[buf].at[pl.ds(ch * GCHUNK, GCHUNK)],
                                        pl.ds(q0, 128)],
                            strips[buf].at[pl.ds(ch * GCHUNK, GCHUNK), pl.ds(0, 128)],
                            sems[buf],
                        ).wait()

            start_pass(0, 0)
            total = jnp.zeros((16,), jnp.float32)
            zeros = jnp.zeros((16,), jnp.float32)
            for psl in range(NPASS):
                if psl + 1 < NPASS:
                    start_pass(psl + 1, (psl + 1) % 2)
                drain_pass(psl % 2)
                buf = psl % 2

                def stat_body(ppos, carry):
                    bsum, bss, asum, ass = carry
                    isaft = ppos >= 25
                    tt = jnp.where(isaft, ppos - 25, ppos)
                    pr = tt // 5
                    pc = tt - pr * 5
                    shift = jnp.where(isaft, 5, 0)
                    nrow = lanes * 10 + pr + shift
                    ncol = lanes * 0 + (j0 + pc + shift)
                    mb = jnp.where(isaft, jnp.float32(0.0), jnp.float32(1.0))
                    ma = jnp.float32(1.0) - mb
                    val = plsc.load_gather(strips[buf], [nrow, ncol])
                    v2 = val * val
                    return (bsum + val * mb, bss + v2 * mb,
                            asum + val * ma, ass + v2 * ma)

                bsum, bss, asum, ass = lax.fori_loop(
                    0, 50, stat_body, (zeros, zeros, zeros, zeros), unroll=5)

                bm = bsum * (1.0 / 25.0)
                am = asum * (1.0 / 25.0)
                bvar = jnp.maximum((bss - bsum * bsum * (1.0 / 25.0)) * (1.0 / 24.0), 0.0)
                avar = jnp.maximum((ass - asum * asum * (1.0 / 25.0)) * (1.0 / 24.0), 0.0)
                bs = _nsqrt(bvar) + 1e-8
                a_s = _nsqrt(avar) + 1e-8
                total = total + jnp.abs(am - bm) / (bs + a_s)

            scorebuf[...] = total
            acc = jnp.zeros((16,), jnp.float32)
            for l in range(16):
                acc = acc + plsc.load_gather(scorebuf, [lanes * 0 + l])
            scorebuf[...] = acc * (1.0 / C) * validf
            pltpu.sync_copy(scorebuf, out_hbm.at[p])
        return 0

    lax.fori_loop(0, 4, pair_body, 0)


def _sc_changepoint(featv, idx):
    mesh = plsc.VectorSubcoreMesh(core_axis_name="c", subcore_axis_name="s")
    kfn = functools.partial(
        pl.kernel,
        _sc_body,
        out_type=jax.ShapeDtypeStruct((128, 16), jnp.float32),
        mesh=mesh,
        compiler_params=pltpu.CompilerParams(needs_layout_passes=False),
        scratch_types=[
            pltpu.VMEM((16,), jnp.int32),
            pltpu.VMEM((ROWS_PER_PASS,), jnp.int32),
            pltpu.VMEM((ROWS_PER_PASS,), jnp.int32),
            pltpu.VMEM((ROWS_PER_PASS, 256), jnp.float32),
            pltpu.VMEM((ROWS_PER_PASS, 256), jnp.float32),
            pltpu.VMEM((16,), jnp.float32),
            pltpu.SemaphoreType.DMA,
            pltpu.SemaphoreType.DMA,
        ],
    )()
    return kfn(featv, idx)


def _combine_body(part_ref, idx_ref, sc_ref, out_ref):
    ic = (part_ref[0, 0, 0] + part_ref[1, 0, 0]) * 0.5
    es = (part_ref[0, 0, 1] + part_ref[1, 0, 1]) * 0.5

    idxv = idx_ref[...]
    kio = lax.broadcasted_iota(jnp.int32, (B, 1, IDX_PAD), 2)
    ii = idxv // W
    jj = idxv - ii * W
    validm = ((kio < K_CP) & (ii >= WIN) & (ii < H - WIN)
              & (jj >= WIN) & (jj < W - WIN)).astype(jnp.float32)
    bio = lax.broadcasted_iota(jnp.int32, (B, 1, IDX_PAD), 0)
    cnt0 = jnp.sum(validm * (bio == 0))
    cnt1 = jnp.sum(validm * (bio == 1))

    col = sc_ref[:, 0:1]
    rio = lax.broadcasted_iota(jnp.int32, (128, 1), 0)
    s0 = jnp.sum(jnp.where(rio < K_CP, col, 0.0))
    s1 = jnp.sum(jnp.where((rio >= K_CP) & (rio < NPAIR), col, 0.0))

    cp0 = jnp.where(cnt0 > 0, s0 / jnp.maximum(cnt0, 1.0), 0.0)
    cp1 = jnp.where(cnt1 > 0, s1 / jnp.maximum(cnt1, 1.0), 0.0)
    cp = (cp0 + cp1) * 0.5
    out_ref[0, 0] = ALPHA * ic - BETA * es + GAMMA * cp


def _combine(part, idx, scores_all):
    return pl.pallas_call(
        _combine_body,
        in_specs=[
            pl.BlockSpec(memory_space=pltpu.SMEM),
            pl.BlockSpec((B, 1, IDX_PAD), lambda: (0, 0, 0)),
            pl.BlockSpec((128, 16), lambda: (0, 0)),
        ],
        out_specs=pl.BlockSpec(memory_space=pltpu.SMEM),
        out_shape=jax.ShapeDtypeStruct((1, 1), jnp.float32),
    )(part, idx, scores_all)


def kernel(segmentation, hic_matrix, features):
    seg = segmentation.reshape(B, H, W)
    hic = hic_matrix.reshape(B, H, W)

    idx3 = _tc_sel(seg)
    featv = features.reshape(-1, W)
    scores_all = _sc_changepoint(featv, idx3.reshape(B, IDX_PAD))
    part = _tc_stats(seg, hic)

    return _combine(part, idx3, scores_all)[0, 0]
